# baseline (speedup 1.0000x reference)
import functools

import jax
import jax.numpy as jnp
from jax import lax
from jax.experimental import pallas as pl
from jax.experimental.pallas import tpu as pltpu
from jax.experimental.pallas import tpu_sc as plsc

_NC = 2
_NS = 16
_NW = _NC * _NS

_BATCH = 4096
_SEQ = 50
_EMB = 64
_BPW = _BATCH // _NW
_ROWS_PER_CHUNK = 1
_IDX_PER_CHUNK = _ROWS_PER_CHUNK * _SEQ
_NCHUNK = _BPW // _ROWS_PER_CHUNK


def _sc_pool(x_hbm, table_hbm, out_hbm, idx_v, buf0, buf1, out_v, sem0, sem1):
    wid = lax.axis_index("s") * _NC + lax.axis_index("c")

    pltpu.sync_copy(x_hbm.at[pl.ds(wid * _BPW, _BPW)], idx_v)

    def start(c, buf, sem):
        pltpu.make_async_copy(table_hbm.at[idx_v.at[c]], buf, sem).start()

    def wait(buf, sem):
        pltpu.make_async_copy(table_hbm.at[idx_v.at[0]], buf, sem).wait()

    def process(c, buf):
        zeros = jnp.zeros((16,), jnp.float32)

        def red_body(s, carry):
            a = list(carry)
            for r in range(_ROWS_PER_CHUNK):
                for d in range(_EMB // 16):
                    a[r * (_EMB // 16) + d] = a[r * (_EMB // 16) + d] + buf[
                        r * _SEQ + s, pl.ds(d * 16, 16)
                    ]
            return tuple(a)

        acc = lax.fori_loop(0, _SEQ, red_body, (zeros,) * (_ROWS_PER_CHUNK * (_EMB // 16)))
        for r in range(_ROWS_PER_CHUNK):
            for d in range(_EMB // 16):
                out_v[c * _ROWS_PER_CHUNK + r, pl.ds(d * 16, 16)] = acc[r * (_EMB // 16) + d]

    start(0, buf0, sem0)

    def outer(g, _):
        start(2 * g + 1, buf1, sem1)
        wait(buf0, sem0)
        process(2 * g, buf0)
        start(2 * g + 2, buf0, sem0)
        wait(buf1, sem1)
        process(2 * g + 1, buf1)
        return 0

    lax.fori_loop(0, _NCHUNK // 2 - 1, outer, 0)
    start(_NCHUNK - 1, buf1, sem1)
    wait(buf0, sem0)
    process(_NCHUNK - 2, buf0)
    wait(buf1, sem1)
    process(_NCHUNK - 1, buf1)

    pltpu.sync_copy(out_v, out_hbm.at[pl.ds(wid * _BPW, _BPW)])


@functools.partial(jax.jit, static_argnums=())
def _pooled_sum(x_r, table):
    mesh = plsc.VectorSubcoreMesh(core_axis_name="c", subcore_axis_name="s")
    return pl.kernel(
        _sc_pool,
        mesh=mesh,
        compiler_params=pltpu.CompilerParams(use_tc_tiling_on_sc=False),
        out_type=jax.ShapeDtypeStruct((_BATCH, _EMB), jnp.float32),
        scratch_types=[
            pltpu.VMEM((_BPW, _SEQ), jnp.int32),
            pltpu.VMEM((_IDX_PER_CHUNK, _EMB), jnp.float32),
            pltpu.VMEM((_IDX_PER_CHUNK, _EMB), jnp.float32),
            pltpu.VMEM((_BPW, _EMB), jnp.float32),
            pltpu.SemaphoreType.DMA,
            pltpu.SemaphoreType.DMA,
        ],
    )(x_r, table)


def _mlp_body(h_ref, w1t_ref, b1_ref, w2t_ref, b2_ref, o_ref):
    h = h_ref[...] * (1.0 / _SEQ)
    a = jnp.dot(h, w1t_ref[...], preferred_element_type=jnp.float32) + b1_ref[...]
    a = jnp.maximum(a, 0.0)
    z = jnp.dot(a, w2t_ref[...], preferred_element_type=jnp.float32) + b2_ref[...]
    o_ref[...] = jax.nn.sigmoid(z)


def kernel(x, table, W1, b1, W2, b2):
    h_sum = _pooled_sum(x.astype(jnp.int32), table)

    out = pl.pallas_call(
        _mlp_body,
        out_shape=jax.ShapeDtypeStruct((_BATCH, 1), jnp.float32),
    )(h_sum, W1.T, b1.reshape(1, 32), W2.T, b2.reshape(1, 1))
    return out.squeeze(-1)


# speedup vs baseline: 1.3229x; 1.3229x over previous
import functools

import jax
import jax.numpy as jnp
from jax import lax
from jax.experimental import pallas as pl
from jax.experimental.pallas import tpu as pltpu
from jax.experimental.pallas import tpu_sc as plsc

_NC = 2
_NS = 16
_NW = _NC * _NS

_BATCH = 4096
_SEQ = 50
_EMB = 64
_NV = _EMB // 16
_BPW = _BATCH // _NW
_RPC = 2
_IPC = _RPC * _SEQ
_NCHUNK = _BPW // _RPC
_NBUF = 4


def _sc_pool(x_hbm, table_hbm, out_hbm, idx_v, b0, b1, b2, b3, out_v, s0, s1, s2, s3):
    bufs = (b0, b1, b2, b3)
    sems = (s0, s1, s2, s3)
    wid = lax.axis_index("s") * _NC + lax.axis_index("c")

    pltpu.sync_copy(x_hbm.at[pl.ds(wid * _BPW, _BPW)], idx_v)

    def start(c, u):
        pltpu.make_async_copy(
            table_hbm.at[idx_v.at[2 * c]], bufs[u].at[pl.ds(0, _SEQ)], sems[u]
        ).start()
        pltpu.make_async_copy(
            table_hbm.at[idx_v.at[2 * c + 1]], bufs[u].at[pl.ds(_SEQ, _SEQ)], sems[u]
        ).start()

    def wait(u):
        pltpu.make_async_copy(table_hbm.at[idx_v.at[0]], bufs[u], sems[u]).wait()

    def process(c, u):
        buf = bufs[u]
        zeros = jnp.zeros((16,), jnp.float32)

        def red_body(s, carry):
            a = list(carry)
            for k in range(2):
                for r in range(_RPC):
                    for d in range(_NV):
                        a[r * _NV + d] = a[r * _NV + d] + buf[
                            r * _SEQ + 2 * s + k, pl.ds(d * 16, 16)
                        ]
            return tuple(a)

        acc = lax.fori_loop(0, _SEQ // 2, red_body, (zeros,) * (_RPC * _NV))
        for r in range(_RPC):
            for d in range(_NV):
                out_v[c * _RPC + r, pl.ds(d * 16, 16)] = acc[r * _NV + d]

    for u in range(_NBUF):
        start(u, u)

    def outer(g, _):
        for u in range(_NBUF):
            wait(u)
            process(_NBUF * g + u, u)
            start(_NBUF * g + u + _NBUF, u)
        return 0

    lax.fori_loop(0, _NCHUNK // _NBUF - 1, outer, 0)
    for u in range(_NBUF):
        wait(u)
        process(_NCHUNK - _NBUF + u, u)

    pltpu.sync_copy(out_v, out_hbm.at[pl.ds(wid * _BPW, _BPW)])


def _pooled_sum(x, table):
    mesh = plsc.VectorSubcoreMesh(core_axis_name="c", subcore_axis_name="s")
    return pl.kernel(
        _sc_pool,
        mesh=mesh,
        compiler_params=pltpu.CompilerParams(use_tc_tiling_on_sc=False),
        out_type=jax.ShapeDtypeStruct((_BATCH, _EMB), jnp.float32),
        scratch_types=[
            pltpu.VMEM((_BPW, _SEQ), jnp.int32),
            pltpu.VMEM((_IPC, _EMB), jnp.float32),
            pltpu.VMEM((_IPC, _EMB), jnp.float32),
            pltpu.VMEM((_IPC, _EMB), jnp.float32),
            pltpu.VMEM((_IPC, _EMB), jnp.float32),
            pltpu.VMEM((_BPW, _EMB), jnp.float32),
            pltpu.SemaphoreType.DMA,
            pltpu.SemaphoreType.DMA,
            pltpu.SemaphoreType.DMA,
            pltpu.SemaphoreType.DMA,
        ],
    )(x, table)


def _mlp_body(h_ref, w1t_ref, b1_ref, w2t_ref, b2_ref, o_ref):
    h = h_ref[...] * (1.0 / _SEQ)
    a = jnp.dot(h, w1t_ref[...], preferred_element_type=jnp.float32) + b1_ref[...]
    a = jnp.maximum(a, 0.0)
    z = jnp.dot(a, w2t_ref[...], preferred_element_type=jnp.float32) + b2_ref[...]
    o_ref[...] = jax.nn.sigmoid(z)


def kernel(x, table, W1, b1, W2, b2):
    h_sum = _pooled_sum(x.astype(jnp.int32), table)

    out = pl.pallas_call(
        _mlp_body,
        out_shape=jax.ShapeDtypeStruct((_BATCH, 1), jnp.float32),
    )(h_sum, W1.T, b1.reshape(1, 32), W2.T, b2.reshape(1, 1))
    return out.squeeze(-1)
